# baseline (speedup 1.0000x reference)
import functools

import jax
import jax.numpy as jnp
from jax import lax
from jax.experimental import pallas as pl
from jax.experimental.pallas import tpu as pltpu
from jax.experimental.pallas import tpu_sc as plsc

_VOCAB = 100000
_DIM = 64
_BATCH = 1024

_NC = 2
_NS = 16
_NW = _NC * _NS
_B_PER_W = _BATCH // _NW
_LANES = 16


def _gather_body(table_hbm, idx_hbm, out_hbm, idx_v, idx2_v, rows_v, sem):
    wid = lax.axis_index("s") * _NC + lax.axis_index("c")
    base = wid * _B_PER_W
    pltpu.sync_copy(idx_hbm.at[pl.ds(base, _B_PER_W)], idx_v)
    for h in range(_B_PER_W // _LANES):
        sl = pl.ds(h * _LANES, _LANES)
        idx2_v[sl] = lax.shift_right_logical(idx_v[sl], 1)
    pltpu.async_copy(table_hbm.at[idx2_v], rows_v, sem).wait()
    pltpu.sync_copy(rows_v, out_hbm.at[pl.ds(base, _B_PER_W)])


def _sc_gather(table2, idx):
    mesh = plsc.VectorSubcoreMesh(core_axis_name="c", subcore_axis_name="s")
    k = functools.partial(
        pl.kernel,
        out_type=jax.ShapeDtypeStruct((_BATCH, 2 * _DIM), jnp.float32),
        mesh=mesh,
        scratch_types=[
            pltpu.VMEM((_B_PER_W,), jnp.int32),
            pltpu.VMEM((_B_PER_W,), jnp.int32),
            pltpu.VMEM((_B_PER_W, 2 * _DIM), jnp.float32),
            pltpu.SemaphoreType.DMA,
        ],
    )(_gather_body)
    return k(table2, idx)


_TILE = 4096


def _proj_body(emb2_ref, idx_ref, wt_ref, b_ref, out_ref, embt_ref):
    @pl.when(pl.program_id(0) == 0)
    def _():
        parity = (idx_ref[...] % 2) == 1
        emb2 = emb2_ref[...]
        emb = jnp.where(parity, emb2[:, _DIM:], emb2[:, :_DIM])
        embt_ref[...] = emb.T

    out_ref[...] = (
        lax.dot_general(
            wt_ref[...],
            embt_ref[...],
            (((0,), (0,)), ((), ())),
            preferred_element_type=jnp.float32,
        )
        + b_ref[...].T
    )


def _projection_t(embeds2, idx2d, w_t, bias_row):
    grid = (pl.cdiv(_VOCAB, _TILE),)
    return pl.pallas_call(
        _proj_body,
        grid=grid,
        in_specs=[
            pl.BlockSpec((_BATCH, 2 * _DIM), lambda j: (0, 0)),
            pl.BlockSpec((_BATCH, 1), lambda j: (0, 0)),
            pl.BlockSpec((_DIM, _TILE), lambda j: (0, j)),
            pl.BlockSpec((1, _TILE), lambda j: (0, j)),
        ],
        out_specs=pl.BlockSpec((_TILE, _BATCH), lambda j: (j, 0)),
        out_shape=jax.ShapeDtypeStruct((_VOCAB, _BATCH), jnp.float32),
        scratch_shapes=[pltpu.VMEM((_DIM, _BATCH), jnp.float32)],
    )(embeds2, idx2d, w_t, bias_row)


def kernel(inputs, embedding_table, linear_w, linear_b):
    idx = inputs.astype(jnp.int32)
    table2 = embedding_table.reshape(_VOCAB // 2, 2 * _DIM)
    embeds2 = _sc_gather(table2, idx)
    out_t = _projection_t(
        embeds2, idx.reshape(_BATCH, 1), linear_w.T, linear_b.reshape(1, _VOCAB)
    )
    return out_t.T


# speedup vs baseline: 1.0574x; 1.0574x over previous
import functools

import jax
import jax.numpy as jnp
from jax import lax
from jax.experimental import pallas as pl
from jax.experimental.pallas import tpu as pltpu
from jax.experimental.pallas import tpu_sc as plsc

_VOCAB = 100000
_DIM = 64
_BATCH = 1024

_RC = 2048


def _relayout_body(tt_ref, out_ref):
    xt = tt_ref[...].T
    out_ref[:, :_DIM] = xt
    out_ref[:, _DIM:] = xt


def _relayout(table_t):
    return pl.pallas_call(
        _relayout_body,
        grid=(pl.cdiv(_VOCAB, _RC),),
        in_specs=[pl.BlockSpec((_DIM, _RC), lambda j: (0, j))],
        out_specs=pl.BlockSpec((_RC, 2 * _DIM), lambda j: (j, 0)),
        out_shape=jax.ShapeDtypeStruct((_VOCAB, 2 * _DIM), jnp.float32),
    )(table_t)


_NC = 2
_NS = 16
_NW = _NC * _NS
_B_PER_W = _BATCH // _NW


def _gather_body(table_hbm, idx_hbm, out_hbm, idx_v, rows_v, sem):
    wid = lax.axis_index("s") * _NC + lax.axis_index("c")
    base = wid * _B_PER_W
    pltpu.sync_copy(idx_hbm.at[pl.ds(base, _B_PER_W)], idx_v)
    pltpu.async_copy(table_hbm.at[idx_v], rows_v, sem).wait()
    pltpu.sync_copy(rows_v, out_hbm.at[pl.ds(base, _B_PER_W)])


def _sc_gather(table3, idx):
    mesh = plsc.VectorSubcoreMesh(core_axis_name="c", subcore_axis_name="s")
    k = functools.partial(
        pl.kernel,
        out_type=jax.ShapeDtypeStruct((_BATCH, 2 * _DIM), jnp.float32),
        mesh=mesh,
        scratch_types=[
            pltpu.VMEM((_B_PER_W,), jnp.int32),
            pltpu.VMEM((_B_PER_W, 2 * _DIM), jnp.float32),
            pltpu.SemaphoreType.DMA,
        ],
    )(_gather_body)
    return k(table3, idx)


_TILE = 4096


def _proj_body(emb2_ref, wt_ref, b_ref, out_ref, embt_ref):
    @pl.when(pl.program_id(0) == 0)
    def _():
        embt_ref[...] = emb2_ref[...].T[:_DIM, :]

    out_ref[...] = (
        lax.dot_general(
            wt_ref[...],
            embt_ref[...],
            (((0,), (0,)), ((), ())),
            preferred_element_type=jnp.float32,
        )
        + b_ref[...].T
    )


def _projection_t(embeds2, w_t, bias_row):
    grid = (pl.cdiv(_VOCAB, _TILE),)
    return pl.pallas_call(
        _proj_body,
        grid=grid,
        in_specs=[
            pl.BlockSpec((_BATCH, 2 * _DIM), lambda j: (0, 0)),
            pl.BlockSpec((_DIM, _TILE), lambda j: (0, j)),
            pl.BlockSpec((1, _TILE), lambda j: (0, j)),
        ],
        out_specs=pl.BlockSpec((_TILE, _BATCH), lambda j: (j, 0)),
        out_shape=jax.ShapeDtypeStruct((_VOCAB, _BATCH), jnp.float32),
        scratch_shapes=[pltpu.VMEM((_DIM, _BATCH), jnp.float32)],
    )(embeds2, w_t, bias_row)


def kernel(inputs, embedding_table, linear_w, linear_b):
    idx = inputs.astype(jnp.int32)
    table3 = _relayout(embedding_table.T)
    embeds2 = _sc_gather(table3, idx)
    out_t = _projection_t(embeds2, linear_w.T, linear_b.reshape(1, _VOCAB))
    return out_t.T


# speedup vs baseline: 1.0738x; 1.0155x over previous
import functools

import jax
import jax.numpy as jnp
from jax import lax
from jax.experimental import pallas as pl
from jax.experimental.pallas import tpu as pltpu
from jax.experimental.pallas import tpu_sc as plsc

_VOCAB = 100000
_DIM = 64
_BATCH = 1024

_RC = 2048


def _relayout_body(tt_ref, out_ref):
    r = lax.broadcasted_iota(jnp.int32, (_DIM, _DIM), 0)
    c = lax.broadcasted_iota(jnp.int32, (_DIM, _DIM), 1)
    eye = (r == c).astype(jnp.float32)
    xt = lax.dot_general(
        tt_ref[...], eye, (((0,), (0,)), ((), ())),
        preferred_element_type=jnp.float32,
    )
    out_ref[:, :_DIM] = xt


def _relayout(table_t):
    return pl.pallas_call(
        _relayout_body,
        grid=(pl.cdiv(_VOCAB, _RC),),
        in_specs=[pl.BlockSpec((_DIM, _RC), lambda j: (0, j))],
        out_specs=pl.BlockSpec((_RC, 2 * _DIM), lambda j: (j, 0)),
        out_shape=jax.ShapeDtypeStruct((_VOCAB, 2 * _DIM), jnp.float32),
    )(table_t)


_NC = 2
_NS = 16
_NW = _NC * _NS
_B_PER_W = _BATCH // _NW


def _gather_body(table_hbm, idx_hbm, out_hbm, idx_v, rows_v, sem):
    wid = lax.axis_index("s") * _NC + lax.axis_index("c")
    base = wid * _B_PER_W
    pltpu.sync_copy(idx_hbm.at[pl.ds(base, _B_PER_W)], idx_v)
    pltpu.async_copy(table_hbm.at[idx_v], rows_v, sem).wait()
    pltpu.sync_copy(rows_v, out_hbm.at[pl.ds(base, _B_PER_W)])


def _sc_gather(table3, idx):
    mesh = plsc.VectorSubcoreMesh(core_axis_name="c", subcore_axis_name="s")
    k = functools.partial(
        pl.kernel,
        out_type=jax.ShapeDtypeStruct((_BATCH, 2 * _DIM), jnp.float32),
        mesh=mesh,
        scratch_types=[
            pltpu.VMEM((_B_PER_W,), jnp.int32),
            pltpu.VMEM((_B_PER_W, 2 * _DIM), jnp.float32),
            pltpu.SemaphoreType.DMA,
        ],
    )(_gather_body)
    return k(table3, idx)


_TILE = 4096


def _proj_body(emb2_ref, wt_ref, b_ref, out_ref, embt_ref):
    @pl.when(pl.program_id(0) == 0)
    def _():
        embt_ref[...] = emb2_ref[...].T[:_DIM, :]

    out_ref[...] = (
        lax.dot_general(
            wt_ref[...],
            embt_ref[...],
            (((0,), (0,)), ((), ())),
            preferred_element_type=jnp.float32,
        )
        + b_ref[...].T
    )


def _projection_t(embeds2, w_t, bias_row):
    grid = (pl.cdiv(_VOCAB, _TILE),)
    return pl.pallas_call(
        _proj_body,
        grid=grid,
        in_specs=[
            pl.BlockSpec((_BATCH, 2 * _DIM), lambda j: (0, 0)),
            pl.BlockSpec((_DIM, _TILE), lambda j: (0, j)),
            pl.BlockSpec((1, _TILE), lambda j: (0, j)),
        ],
        out_specs=pl.BlockSpec((_TILE, _BATCH), lambda j: (j, 0)),
        out_shape=jax.ShapeDtypeStruct((_VOCAB, _BATCH), jnp.float32),
        scratch_shapes=[pltpu.VMEM((_DIM, _BATCH), jnp.float32)],
    )(embeds2, w_t, bias_row)


def kernel(inputs, embedding_table, linear_w, linear_b):
    idx = inputs.astype(jnp.int32)
    table3 = _relayout(embedding_table.T)
    embeds2 = _sc_gather(table3, idx)
    out_t = _projection_t(embeds2, linear_w.T, linear_b.reshape(1, _VOCAB))
    return out_t.T


# speedup vs baseline: 1.2090x; 1.1259x over previous
import functools

import jax
import jax.numpy as jnp
from jax import lax
from jax.experimental import pallas as pl
from jax.experimental.pallas import tpu as pltpu
from jax.experimental.pallas import tpu_sc as plsc

_VOCAB = 100000
_DIM = 64
_BATCH = 1024

_RC = 16384


def _relayout_body(tt_ref, out_ref):
    r = lax.broadcasted_iota(jnp.int32, (_DIM, _DIM), 0)
    c = lax.broadcasted_iota(jnp.int32, (_DIM, _DIM), 1)
    eye = (r == c).astype(jnp.float32)
    xt = lax.dot_general(
        tt_ref[...], eye, (((0,), (0,)), ((), ())),
        preferred_element_type=jnp.float32,
    )
    out_ref[:, :_DIM] = xt


def _relayout(table_t):
    return pl.pallas_call(
        _relayout_body,
        grid=(pl.cdiv(_VOCAB, _RC),),
        in_specs=[pl.BlockSpec((_DIM, _RC), lambda j: (0, j))],
        out_specs=pl.BlockSpec((_RC, 2 * _DIM), lambda j: (j, 0)),
        out_shape=jax.ShapeDtypeStruct((_VOCAB, 2 * _DIM), jnp.float32),
    )(table_t)


_NC = 2
_NS = 16
_NW = _NC * _NS
_B_PER_W = _BATCH // _NW


def _gather_body(table_hbm, idx_hbm, out_hbm, idx_v, rows_v, sem):
    wid = lax.axis_index("s") * _NC + lax.axis_index("c")
    base = wid * _B_PER_W
    pltpu.sync_copy(idx_hbm.at[pl.ds(base, _B_PER_W)], idx_v)
    pltpu.async_copy(table_hbm.at[idx_v], rows_v, sem).wait()
    pltpu.sync_copy(rows_v, out_hbm.at[pl.ds(base, _B_PER_W)])


def _sc_gather(table3, idx):
    mesh = plsc.VectorSubcoreMesh(core_axis_name="c", subcore_axis_name="s")
    k = functools.partial(
        pl.kernel,
        out_type=jax.ShapeDtypeStruct((_BATCH, 2 * _DIM), jnp.float32),
        mesh=mesh,
        scratch_types=[
            pltpu.VMEM((_B_PER_W,), jnp.int32),
            pltpu.VMEM((_B_PER_W, 2 * _DIM), jnp.float32),
            pltpu.SemaphoreType.DMA,
        ],
    )(_gather_body)
    return k(table3, idx)


_TILE = 4096


def _proj_body(emb2_ref, wt_ref, b_ref, out_ref, embt_ref):
    @pl.when(pl.program_id(0) == 0)
    def _():
        embt_ref[...] = emb2_ref[...].T[:_DIM, :]

    out_ref[...] = (
        lax.dot_general(
            wt_ref[...],
            embt_ref[...],
            (((0,), (0,)), ((), ())),
            preferred_element_type=jnp.float32,
        )
        + b_ref[...].T
    )


def _projection_t(embeds2, w_t, bias_row):
    grid = (pl.cdiv(_VOCAB, _TILE),)
    return pl.pallas_call(
        _proj_body,
        grid=grid,
        in_specs=[
            pl.BlockSpec((_BATCH, 2 * _DIM), lambda j: (0, 0)),
            pl.BlockSpec((_DIM, _TILE), lambda j: (0, j)),
            pl.BlockSpec((1, _TILE), lambda j: (0, j)),
        ],
        out_specs=pl.BlockSpec((_TILE, _BATCH), lambda j: (j, 0)),
        out_shape=jax.ShapeDtypeStruct((_VOCAB, _BATCH), jnp.float32),
        scratch_shapes=[pltpu.VMEM((_DIM, _BATCH), jnp.float32)],
    )(embeds2, w_t, bias_row)


def kernel(inputs, embedding_table, linear_w, linear_b):
    idx = inputs.astype(jnp.int32)
    table3 = _relayout(embedding_table.T)
    embeds2 = _sc_gather(table3, idx)
    out_t = _projection_t(embeds2, linear_w.T, linear_b.reshape(1, _VOCAB))
    return out_t.T


# speedup vs baseline: 1.2127x; 1.0031x over previous
import functools

import jax
import jax.numpy as jnp
from jax import lax
from jax.experimental import pallas as pl
from jax.experimental.pallas import tpu as pltpu
from jax.experimental.pallas import tpu_sc as plsc

_VOCAB = 100000
_DIM = 64
_BATCH = 1024

_RC = 32768


def _relayout_body(tt_ref, out_ref):
    r = lax.broadcasted_iota(jnp.int32, (_DIM, _DIM), 0)
    c = lax.broadcasted_iota(jnp.int32, (_DIM, _DIM), 1)
    eye = (r == c).astype(jnp.float32)
    xt = lax.dot_general(
        tt_ref[...], eye, (((0,), (0,)), ((), ())),
        preferred_element_type=jnp.float32,
    )
    out_ref[:, :_DIM] = xt


def _relayout(table_t):
    return pl.pallas_call(
        _relayout_body,
        grid=(pl.cdiv(_VOCAB, _RC),),
        in_specs=[pl.BlockSpec((_DIM, _RC), lambda j: (0, j))],
        out_specs=pl.BlockSpec((_RC, 2 * _DIM), lambda j: (j, 0)),
        out_shape=jax.ShapeDtypeStruct((_VOCAB, 2 * _DIM), jnp.float32),
    )(table_t)


_NC = 2
_NS = 16
_NW = _NC * _NS
_B_PER_W = _BATCH // _NW


def _gather_body(table_hbm, idx_hbm, out_hbm, idx_v, rows_v, sem):
    wid = lax.axis_index("s") * _NC + lax.axis_index("c")
    base = wid * _B_PER_W
    pltpu.sync_copy(idx_hbm.at[pl.ds(base, _B_PER_W)], idx_v)
    pltpu.async_copy(table_hbm.at[idx_v], rows_v, sem).wait()
    pltpu.sync_copy(rows_v, out_hbm.at[pl.ds(base, _B_PER_W)])


def _sc_gather(table3, idx):
    mesh = plsc.VectorSubcoreMesh(core_axis_name="c", subcore_axis_name="s")
    k = functools.partial(
        pl.kernel,
        out_type=jax.ShapeDtypeStruct((_BATCH, 2 * _DIM), jnp.float32),
        mesh=mesh,
        scratch_types=[
            pltpu.VMEM((_B_PER_W,), jnp.int32),
            pltpu.VMEM((_B_PER_W, 2 * _DIM), jnp.float32),
            pltpu.SemaphoreType.DMA,
        ],
    )(_gather_body)
    return k(table3, idx)


_TILE = 4096


def _proj_body(emb2_ref, wt_ref, b_ref, out_ref, embt_ref):
    @pl.when(pl.program_id(0) == 0)
    def _():
        embt_ref[...] = emb2_ref[...].T[:_DIM, :]

    out_ref[...] = (
        lax.dot_general(
            wt_ref[...],
            embt_ref[...],
            (((0,), (0,)), ((), ())),
            preferred_element_type=jnp.float32,
        )
        + b_ref[...].T
    )


def _projection_t(embeds2, w_t, bias_row):
    grid = (pl.cdiv(_VOCAB, _TILE),)
    return pl.pallas_call(
        _proj_body,
        grid=grid,
        in_specs=[
            pl.BlockSpec((_BATCH, 2 * _DIM), lambda j: (0, 0)),
            pl.BlockSpec((_DIM, _TILE), lambda j: (0, j)),
            pl.BlockSpec((1, _TILE), lambda j: (0, j)),
        ],
        out_specs=pl.BlockSpec((_TILE, _BATCH), lambda j: (j, 0)),
        out_shape=jax.ShapeDtypeStruct((_VOCAB, _BATCH), jnp.float32),
        scratch_shapes=[pltpu.VMEM((_DIM, _BATCH), jnp.float32)],
    )(embeds2, w_t, bias_row)


def kernel(inputs, embedding_table, linear_w, linear_b):
    idx = inputs.astype(jnp.int32)
    table3 = _relayout(embedding_table.T)
    embeds2 = _sc_gather(table3, idx)
    out_t = _projection_t(embeds2, linear_w.T, linear_b.reshape(1, _VOCAB))
    return out_t.T
